# Initial kernel scaffold; baseline (speedup 1.0000x reference)
import jax
import jax.numpy as jnp
from jax.experimental import pallas as pl


def kernel(x, ylr, msk_lr, gt, gy, gx, st, sy, sx, sv, nt, ny, nx, nv):
    raise NotImplementedError("write your pallas kernel here")


# speedup vs baseline: 1.0612x; 1.0612x over previous
import functools

import jax
import jax.numpy as jnp
from jax import lax
from jax.experimental import pallas as pl
from jax.experimental.pallas import tpu as pltpu
from jax.experimental.pallas import tpu_sc as plsc

_DT = 7
_NW = 32
_L = 16


def _dense_body(xlr_ref, xan_ref, ylr_ref, msk_ref, dy_ref, ga_ref):
    xlr = xlr_ref[...]
    dy_ref[...] = (ylr_ref[...] - xlr) * msk_ref[...]
    ga_ref[...] = xlr + xan_ref[...]


def _dense_stage(x, ylr, msk_lr):
    B, T2, H, W = x.shape
    T = T2 // 2
    grid = (B * T,)
    bs = (1, 1, H, W)
    lr_spec = pl.BlockSpec(bs, lambda i: (i // T, i % T, 0, 0))
    an_spec = pl.BlockSpec(bs, lambda i: (i // T, T + i % T, 0, 0))
    o_spec = pl.BlockSpec(bs, lambda i: (i // T, i % T, 0, 0))
    out_shape = [
        jax.ShapeDtypeStruct((B, T, H, W), jnp.float32),
        jax.ShapeDtypeStruct((B, T, H, W), jnp.float32),
    ]
    return pl.pallas_call(
        _dense_body,
        grid=grid,
        in_specs=[lr_spec, an_spec, o_spec, o_spec],
        out_specs=[o_spec, o_spec],
        out_shape=out_shape,
    )(x, x, ylr, msk_lr)


def _interp_stage(wid, table_ref, st_h, sy_h, sx_h, sv_h, out_h,
                  c_st, c_sy, c_sx, c_sv, wbuf, idxbuf, gbuf, obuf,
                  pbuf, T, H, W, C, cpb, nch, tsc_row):
    NG = C // _L
    gt0 = pbuf[0]
    tsc = pbuf[tsc_row]
    gy0 = pbuf[2]
    ysc = pbuf[3]
    gx0 = pbuf[4]
    xsc = pbuf[5]
    tstride = T * H * W

    def chunk_body(i, _):
        cid = wid + i * _NW
        base = cid * C
        tb = (cid // cpb) * tstride
        pltpu.sync_copy(st_h.at[pl.ds(base, C)], c_st)
        pltpu.sync_copy(sy_h.at[pl.ds(base, C)], c_sy)
        pltpu.sync_copy(sx_h.at[pl.ds(base, C)], c_sx)
        pltpu.sync_copy(sv_h.at[pl.ds(base, C)], c_sv)

        def idx_pass(g, _):
            o = g * _L
            ti = (c_st[pl.ds(o, _L)] - gt0) * tsc
            yi = (c_sy[pl.ds(o, _L)] - gy0) * ysc
            xi = (c_sx[pl.ds(o, _L)] - gx0) * xsc
            valid = ((ti >= 0.0) & (ti <= T - 1.0)
                     & (yi >= 0.0) & (yi <= H - 1.0)
                     & (xi >= 0.0) & (xi <= W - 1.0))
            t0 = jnp.clip(ti, 0.0, T - 2.0).astype(jnp.int32)
            y0 = jnp.clip(yi, 0.0, H - 2.0).astype(jnp.int32)
            x0 = jnp.clip(xi, 0.0, W - 2.0).astype(jnp.int32)
            wt = jnp.clip(ti - t0.astype(jnp.float32), 0.0, 1.0)
            wy = jnp.clip(yi - y0.astype(jnp.float32), 0.0, 1.0)
            wx = jnp.clip(xi - x0.astype(jnp.float32), 0.0, 1.0)
            base000 = tb + (t0 * H + y0) * W + x0
            idxbuf[pl.ds(0 * C + o, _L)] = base000
            idxbuf[pl.ds(1 * C + o, _L)] = base000 + 1
            idxbuf[pl.ds(2 * C + o, _L)] = base000 + W
            idxbuf[pl.ds(3 * C + o, _L)] = base000 + (W + 1)
            idxbuf[pl.ds(4 * C + o, _L)] = base000 + H * W
            idxbuf[pl.ds(5 * C + o, _L)] = base000 + (H * W + 1)
            idxbuf[pl.ds(6 * C + o, _L)] = base000 + (H * W + W)
            idxbuf[pl.ds(7 * C + o, _L)] = base000 + (H * W + W + 1)
            wbuf[pl.ds(0 * C + o, _L)] = wt
            wbuf[pl.ds(1 * C + o, _L)] = wy
            wbuf[pl.ds(2 * C + o, _L)] = wx
            wbuf[pl.ds(3 * C + o, _L)] = jnp.where(valid, 1.0, 0.0)
            return 0

        lax.fori_loop(0, NG, idx_pass, 0)
        pltpu.sync_copy(table_ref.at[idxbuf], gbuf)

        def mix_pass(g, _):
            o = g * _L
            wt = wbuf[pl.ds(0 * C + o, _L)]
            wy = wbuf[pl.ds(1 * C + o, _L)]
            wx = wbuf[pl.ds(2 * C + o, _L)]
            vld = wbuf[pl.ds(3 * C + o, _L)]
            v000 = gbuf[pl.ds(0 * C + o, _L)]
            v001 = gbuf[pl.ds(1 * C + o, _L)]
            v010 = gbuf[pl.ds(2 * C + o, _L)]
            v011 = gbuf[pl.ds(3 * C + o, _L)]
            v100 = gbuf[pl.ds(4 * C + o, _L)]
            v101 = gbuf[pl.ds(5 * C + o, _L)]
            v110 = gbuf[pl.ds(6 * C + o, _L)]
            v111 = gbuf[pl.ds(7 * C + o, _L)]
            c00 = v000 * (1.0 - wt) + v100 * wt
            c01 = v001 * (1.0 - wt) + v101 * wt
            c10 = v010 * (1.0 - wt) + v110 * wt
            c11 = v011 * (1.0 - wt) + v111 * wt
            c0 = c00 * (1.0 - wx) + c01 * wx
            c1 = c10 * (1.0 - wx) + c11 * wx
            sx_val = c0 * (1.0 - wy) + c1 * wy
            res = (sx_val - c_sv[pl.ds(o, _L)]) * vld
            obuf[pl.ds(o, _L)] = res
            return 0

        lax.fori_loop(0, NG, mix_pass, 0)
        pltpu.sync_copy(obuf, out_h.at[pl.ds(base, C)])
        return 0

    count = (nch - wid + (_NW - 1)) // _NW
    lax.fori_loop(0, count, chunk_body, 0)


def _make_sc_kernel(B, T2, H, W, NT, NXs, NN, CS, CN):
    T = T2 // 2
    NS = B * NT * NXs
    NNF = B * NN

    mesh = plsc.VectorSubcoreMesh(core_axis_name="c", subcore_axis_name="s")

    @functools.partial(
        pl.kernel,
        out_type=[
            jax.ShapeDtypeStruct((NS,), jnp.float32),
            jax.ShapeDtypeStruct((NNF,), jnp.float32),
        ],
        mesh=mesh,
        scratch_types=[
            pltpu.VMEM((CS,), jnp.float32),
            pltpu.VMEM((CS,), jnp.float32),
            pltpu.VMEM((CS,), jnp.float32),
            pltpu.VMEM((CS,), jnp.float32),
            pltpu.VMEM((4 * CS,), jnp.float32),
            pltpu.VMEM((8 * CS,), jnp.int32),
            pltpu.VMEM((8 * CS,), jnp.float32),
            pltpu.VMEM((CS,), jnp.float32),
            pltpu.VMEM((CN,), jnp.float32),
            pltpu.VMEM((CN,), jnp.float32),
            pltpu.VMEM((CN,), jnp.float32),
            pltpu.VMEM((CN,), jnp.float32),
            pltpu.VMEM((4 * CN,), jnp.float32),
            pltpu.VMEM((8 * CN,), jnp.int32),
            pltpu.VMEM((8 * CN,), jnp.float32),
            pltpu.VMEM((CN,), jnp.float32),
            pltpu.VMEM((8, _L), jnp.float32),
        ],
    )
    def sc_kernel(tableA_h, tableX_h, st_h, sy_h, sx_h, sv_h,
                  nt_h, ny_h, nx_h, nv_h, params_h,
                  dyout_h, dyout1_h,
                  s_st, s_sy, s_sx, s_sv, s_w, s_idx, s_g, s_o,
                  n_st, n_sy, n_sx, n_sv, n_w, n_idx, n_g, n_o,
                  pbuf):
        wid = lax.axis_index("s") * 2 + lax.axis_index("c")
        pltpu.sync_copy(params_h, pbuf)
        _interp_stage(wid, tableA_h, st_h, sy_h, sx_h, sv_h, dyout_h,
                      s_st, s_sy, s_sx, s_sv, s_w, s_idx, s_g, s_o,
                      pbuf, T, H, W, CS, (NT * NXs) // CS, NS // CS, 1)
        _interp_stage(wid, tableX_h, nt_h, ny_h, nx_h, nv_h, dyout1_h,
                      n_st, n_sy, n_sx, n_sv, n_w, n_idx, n_g, n_o,
                      pbuf, T2, H, W, CN, NN // CN, NNF // CN, 6)

    return sc_kernel


def kernel(x, ylr, msk_lr, gt, gy, gx, st, sy, sx, sv, nt, ny, nx, nv):
    B, T2, H, W = x.shape
    T = T2 // 2
    _, NT, NXs = st.shape
    NN = nt.shape[1]

    dyoutlr, gridA = _dense_stage(x, ylr, msk_lr)

    tden = gt[-1] - gt[0]
    params = jnp.stack([
        gt[0], (T - 1.0) / tden, gy[0], 1.0 / (gy[1] - gy[0]),
        gx[0], 1.0 / (gx[1] - gx[0]), (T2 - 1.0) / tden, 0.0 * gt[0],
    ]).astype(jnp.float32)
    params = jnp.broadcast_to(params[:, None], (8, _L))

    CS, CN = 320, 400
    sc = _make_sc_kernel(B, T2, H, W, NT, NXs, NN, CS, CN)
    dyout_flat, dyout1_flat = sc(
        gridA.reshape(-1), x.reshape(-1),
        st.reshape(-1), sy.reshape(-1), sx.reshape(-1), sv.reshape(-1),
        nt.reshape(-1), ny.reshape(-1), nx.reshape(-1), nv.reshape(-1),
        params)

    return (dyoutlr, dyout_flat.reshape(B, NT, NXs), dyout1_flat.reshape(B, NN))


# speedup vs baseline: 1.5477x; 1.4584x over previous
import functools

import jax
import jax.numpy as jnp
from jax import lax
from jax.experimental import pallas as pl
from jax.experimental.pallas import tpu as pltpu
from jax.experimental.pallas import tpu_sc as plsc

_DT = 7
_NW = 32
_L = 16


def _dense_body(xlr_ref, xan_ref, ylr_ref, msk_ref, dy_ref, ga_ref):
    xlr = xlr_ref[...]
    dy_ref[...] = (ylr_ref[...] - xlr) * msk_ref[...]
    ga_ref[...] = xlr + xan_ref[...]


def _dense_stage(x, ylr, msk_lr):
    B, T2, H, W = x.shape
    T = T2 // 2
    grid = (B * T,)
    bs = (1, 1, H, W)
    lr_spec = pl.BlockSpec(bs, lambda i: (i // T, i % T, 0, 0))
    an_spec = pl.BlockSpec(bs, lambda i: (i // T, T + i % T, 0, 0))
    o_spec = pl.BlockSpec(bs, lambda i: (i // T, i % T, 0, 0))
    out_shape = [
        jax.ShapeDtypeStruct((B, T, H, W), jnp.float32),
        jax.ShapeDtypeStruct((B, T, H, W), jnp.float32),
    ]
    return pl.pallas_call(
        _dense_body,
        grid=grid,
        in_specs=[lr_spec, an_spec, o_spec, o_spec],
        out_specs=[o_spec, o_spec],
        out_shape=out_shape,
    )(x, x, ylr, msk_lr)


def _interp_stage(wid, table_ref, st_h, sy_h, sx_h, sv_h, out_h,
                  bufs, sems, pbuf, T, H, W, C, cpb, nch, tsc_row):
    cst, csy, csx, csv, wbuf, idxbuf, gbuf, obuf = bufs
    sem_c, sem_v, sem_g, sem_o = sems
    NG = C // _L
    nb = table_ref.shape[0] // (T * H * W)
    gt0 = pbuf[0]
    tsc = pbuf[tsc_row]
    gy0 = pbuf[2]
    ysc = pbuf[3]
    gx0 = pbuf[4]
    xsc = pbuf[5]
    tstride = T * H * W
    count = nch // _NW
    assert count >= 2 and count % 2 == 0

    def base_of(k):
        return (wid + k * _NW) * C

    def fire_cxy(k, b):
        base = base_of(k)
        pltpu.async_copy(st_h.at[pl.ds(base, C)], cst[b], sem_c[b])
        pltpu.async_copy(sy_h.at[pl.ds(base, C)], csy[b], sem_c[b])
        pltpu.async_copy(sx_h.at[pl.ds(base, C)], csx[b], sem_c[b])

    def fire_sv(k, b):
        pltpu.async_copy(sv_h.at[pl.ds(base_of(k), C)], csv[b], sem_v[b])

    def do_idx(k, b):
        pltpu.make_async_copy(st_h.at[pl.ds(0, C)], cst[b], sem_c[b]).wait()
        pltpu.make_async_copy(sy_h.at[pl.ds(0, C)], csy[b], sem_c[b]).wait()
        pltpu.make_async_copy(sx_h.at[pl.ds(0, C)], csx[b], sem_c[b]).wait()
        cid = wid + k * _NW
        tb = jnp.minimum(cid // cpb, nb - 1) * tstride

        def idx_pass(g, _):
            o = g * _L
            ti = (cst[b][pl.ds(o, _L)] - gt0) * tsc
            yi = (csy[b][pl.ds(o, _L)] - gy0) * ysc
            xi = (csx[b][pl.ds(o, _L)] - gx0) * xsc
            valid = ((ti >= 0.0) & (ti <= T - 1.0)
                     & (yi >= 0.0) & (yi <= H - 1.0)
                     & (xi >= 0.0) & (xi <= W - 1.0))
            t0 = jnp.clip(ti, 0.0, T - 2.0).astype(jnp.int32)
            y0 = jnp.clip(yi, 0.0, H - 2.0).astype(jnp.int32)
            x0 = jnp.clip(xi, 0.0, W - 2.0).astype(jnp.int32)
            wt = jnp.clip(ti - t0.astype(jnp.float32), 0.0, 1.0)
            wy = jnp.clip(yi - y0.astype(jnp.float32), 0.0, 1.0)
            wx = jnp.clip(xi - x0.astype(jnp.float32), 0.0, 1.0)
            base000 = tb + (t0 * H + y0) * W + x0
            ib = idxbuf[b]
            ib[pl.ds(0 * C + o, _L)] = base000
            ib[pl.ds(1 * C + o, _L)] = base000 + 1
            ib[pl.ds(2 * C + o, _L)] = base000 + W
            ib[pl.ds(3 * C + o, _L)] = base000 + (W + 1)
            ib[pl.ds(4 * C + o, _L)] = base000 + H * W
            ib[pl.ds(5 * C + o, _L)] = base000 + (H * W + 1)
            ib[pl.ds(6 * C + o, _L)] = base000 + (H * W + W)
            ib[pl.ds(7 * C + o, _L)] = base000 + (H * W + W + 1)
            wb = wbuf[b]
            wb[pl.ds(0 * C + o, _L)] = wt
            wb[pl.ds(1 * C + o, _L)] = wy
            wb[pl.ds(2 * C + o, _L)] = wx
            wb[pl.ds(3 * C + o, _L)] = jnp.where(valid, 1.0, 0.0)
            return 0

        lax.fori_loop(0, NG, idx_pass, 0)
        pltpu.async_copy(table_ref.at[idxbuf[b]], gbuf[b], sem_g[b])

    def do_mix(k, b, wait_out):
        pltpu.make_async_copy(table_ref.at[idxbuf[b]], gbuf[b],
                              sem_g[b]).wait()
        pltpu.make_async_copy(sv_h.at[pl.ds(0, C)], csv[b], sem_v[b]).wait()
        if wait_out is not None:
            def _w():
                pltpu.make_async_copy(
                    obuf[b], out_h.at[pl.ds(0, C)], sem_o[b]).wait()
            if wait_out is True:
                _w()
            else:
                pl.when(wait_out)(_w)

        def mix_pass(g, _):
            o = g * _L
            wb = wbuf[b]
            gb = gbuf[b]
            wt = wb[pl.ds(0 * C + o, _L)]
            wy = wb[pl.ds(1 * C + o, _L)]
            wx = wb[pl.ds(2 * C + o, _L)]
            vld = wb[pl.ds(3 * C + o, _L)]
            c00 = gb[pl.ds(0 * C + o, _L)] * (1.0 - wt) \
                + gb[pl.ds(4 * C + o, _L)] * wt
            c01 = gb[pl.ds(1 * C + o, _L)] * (1.0 - wt) \
                + gb[pl.ds(5 * C + o, _L)] * wt
            c10 = gb[pl.ds(2 * C + o, _L)] * (1.0 - wt) \
                + gb[pl.ds(6 * C + o, _L)] * wt
            c11 = gb[pl.ds(3 * C + o, _L)] * (1.0 - wt) \
                + gb[pl.ds(7 * C + o, _L)] * wt
            c0 = c00 * (1.0 - wx) + c01 * wx
            c1 = c10 * (1.0 - wx) + c11 * wx
            sx_val = c0 * (1.0 - wy) + c1 * wy
            obuf[b][pl.ds(o, _L)] = (sx_val - csv[b][pl.ds(o, _L)]) * vld
            return 0

        lax.fori_loop(0, NG, mix_pass, 0)
        pltpu.async_copy(obuf[b], out_h.at[pl.ds(base_of(k), C)], sem_o[b])

    fire_cxy(0, 0)
    fire_sv(0, 0)
    fire_cxy(1, 1)
    fire_sv(1, 1)
    do_idx(0, 0)

    if count > 2:
        def pair(j, _):
            k = 2 * j
            fire_cxy(k + 2, 0)
            do_idx(k + 1, 1)
            do_mix(k, 0, wait_out=(j > 0))
            fire_sv(k + 2, 0)
            fire_cxy(k + 3, 1)
            do_idx(k + 2, 0)
            do_mix(k + 1, 1, wait_out=(j > 0))
            fire_sv(k + 3, 1)
            return 0

        lax.fori_loop(0, (count - 2) // 2, pair, 0)

    tail_wait = True if count > 2 else None
    do_idx(count - 1, 1)
    do_mix(count - 2, 0, wait_out=tail_wait)
    do_mix(count - 1, 1, wait_out=tail_wait)
    pltpu.make_async_copy(obuf[0], out_h.at[pl.ds(0, C)], sem_o[0]).wait()
    pltpu.make_async_copy(obuf[1], out_h.at[pl.ds(0, C)], sem_o[1]).wait()


def _make_sc_kernel(B, T2, H, W, NSP, NNP, CS, CN, cpb_s, cpb_n):
    T = T2 // 2

    mesh = plsc.VectorSubcoreMesh(core_axis_name="c", subcore_axis_name="s")

    def _stage_bufs(C):
        return ([pltpu.VMEM((C,), jnp.float32) for _ in range(2)]
                + [pltpu.VMEM((C,), jnp.float32) for _ in range(2)]
                + [pltpu.VMEM((C,), jnp.float32) for _ in range(2)]
                + [pltpu.VMEM((C,), jnp.float32) for _ in range(2)]
                + [pltpu.VMEM((4 * C,), jnp.float32) for _ in range(2)]
                + [pltpu.VMEM((8 * C,), jnp.int32) for _ in range(2)]
                + [pltpu.VMEM((8 * C,), jnp.float32) for _ in range(2)]
                + [pltpu.VMEM((C,), jnp.float32) for _ in range(2)])

    @functools.partial(
        pl.kernel,
        out_type=[
            jax.ShapeDtypeStruct((NSP,), jnp.float32),
            jax.ShapeDtypeStruct((NNP,), jnp.float32),
        ],
        mesh=mesh,
        scratch_types=(
            _stage_bufs(CS) + _stage_bufs(CN)
            + [pltpu.VMEM((8, _L), jnp.float32)]
            + [pltpu.SemaphoreType.DMA for _ in range(8)]
        ),
    )
    def sc_kernel(tableA_h, tableX_h, st_h, sy_h, sx_h, sv_h,
                  nt_h, ny_h, nx_h, nv_h, params_h,
                  dyout_h, dyout1_h, *scr):
        sbufs = [(scr[2 * i], scr[2 * i + 1]) for i in range(8)]
        nbufs = [(scr[16 + 2 * i], scr[16 + 2 * i + 1]) for i in range(8)]
        pbuf = scr[32]
        sems = [(scr[33 + 2 * i], scr[34 + 2 * i]) for i in range(4)]
        wid = lax.axis_index("s") * 2 + lax.axis_index("c")
        pltpu.sync_copy(params_h, pbuf)
        _interp_stage(wid, tableA_h, st_h, sy_h, sx_h, sv_h, dyout_h,
                      sbufs, sems, pbuf, T, H, W, CS, cpb_s,
                      NSP // CS, 1)
        _interp_stage(wid, tableX_h, nt_h, ny_h, nx_h, nv_h, dyout1_h,
                      nbufs, sems, pbuf, T2, H, W, CN, cpb_n,
                      NNP // CN, 6)

    return sc_kernel


def _pad_to(a, n):
    return jnp.pad(a.reshape(-1), (0, n - a.size))


def kernel(x, ylr, msk_lr, gt, gy, gx, st, sy, sx, sv, nt, ny, nx, nv):
    B, T2, H, W = x.shape
    T = T2 // 2
    _, NT, NXs = st.shape
    NN = nt.shape[1]
    NS = B * NT * NXs
    NNF = B * NN

    dyoutlr, gridA = _dense_stage(x, ylr, msk_lr)

    tden = gt[-1] - gt[0]
    params = jnp.stack([
        gt[0], (T - 1.0) / tden, gy[0], 1.0 / (gy[1] - gy[0]),
        gx[0], 1.0 / (gx[1] - gx[0]), (T2 - 1.0) / tden, 0.0 * gt[0],
    ]).astype(jnp.float32)
    params = jnp.broadcast_to(params[:, None], (8, _L))

    CS, CN = 320, 400
    ppb_s = NT * NXs
    assert ppb_s % CS == 0
    nch_s = -(-NS // CS)
    nch_s += (-nch_s) % (2 * _NW)
    NSP = nch_s * CS
    nch_n = -(-NNF // CN)
    nch_n += (-nch_n) % (2 * _NW)
    NNP = nch_n * CN

    sc = _make_sc_kernel(B, T2, H, W, NSP, NNP, CS, CN,
                         ppb_s // CS, NN // CN)
    dyout_flat, dyout1_flat = sc(
        gridA.reshape(-1), x.reshape(-1),
        _pad_to(st, NSP), _pad_to(sy, NSP), _pad_to(sx, NSP),
        _pad_to(sv, NSP),
        _pad_to(nt, NNP), _pad_to(ny, NNP), _pad_to(nx, NNP),
        _pad_to(nv, NNP),
        params)

    return (dyoutlr,
            dyout_flat[:NS].reshape(B, NT, NXs),
            dyout1_flat[:NNF].reshape(B, NN))
